# baseline (speedup 1.0000x reference)
import functools

import jax
import jax.numpy as jnp
from jax import lax
from jax.experimental import pallas as pl
from jax.experimental.pallas import tpu as pltpu
from jax.experimental.pallas import tpu_sc as plsc

_BATCH = 4096
_HIST = 200
_D = 64
_NC = 2
_NS = 16
_NW = _NC * _NS
_B = _BATCH * _HIST
_B_PER_W = _B // _NW
_IDX_MINOR = 128
_CHUNK = 512
_IDX_ROWS = _CHUNK // _IDX_MINOR
_G = _B_PER_W // _CHUNK

_mesh = plsc.VectorSubcoreMesh(core_axis_name="c", subcore_axis_name="s")


@functools.partial(
    pl.kernel,
    mesh=_mesh,
    out_type=jax.ShapeDtypeStruct((_NW, _G, _CHUNK, _D), jnp.float32),
    scratch_types=[
        pltpu.VMEM((_IDX_ROWS, _IDX_MINOR), jnp.int32),
        pltpu.VMEM((_CHUNK, _D), jnp.float32),
        pltpu.SemaphoreType.DMA,
    ],
    compiler_params=pltpu.CompilerParams(use_tc_tiling_on_sc=False),
)
def _gather_kernel(table_hbm, idx_hbm, out_hbm, idx_v, rows_v, sem):
    wid = lax.axis_index("s") * _NC + lax.axis_index("c")

    def body(g, carry):
        pltpu.sync_copy(idx_hbm.at[wid, g], idx_v)
        handles = []
        for j in range(_IDX_ROWS):
            handles.append(
                pltpu.async_copy(
                    table_hbm.at[idx_v.at[j]],
                    rows_v.at[pl.ds(j * _IDX_MINOR, _IDX_MINOR)],
                    sem,
                )
            )
        for h in handles:
            h.wait()
        pltpu.sync_copy(rows_v, out_hbm.at[wid, g])
        return carry

    lax.fori_loop(0, _G, body, 0)


def kernel(item_seq, ID_embeddings):
    idx = item_seq.astype(jnp.int32).reshape(_NW, _G, _IDX_ROWS, _IDX_MINOR)
    out = _gather_kernel(ID_embeddings, idx)
    return out.reshape(_BATCH, _HIST, _D)


# speedup vs baseline: 1.0429x; 1.0429x over previous
import functools

import jax
import jax.numpy as jnp
from jax import lax
from jax.experimental import pallas as pl
from jax.experimental.pallas import tpu as pltpu
from jax.experimental.pallas import tpu_sc as plsc

_BATCH = 4096
_HIST = 200
_D = 64
_NC = 2
_NS = 16
_NW = _NC * _NS
_B = _BATCH * _HIST
_B_PER_W = _B // _NW
_IDX_MINOR = 128
_IDX_ROWS_W = _B_PER_W // _IDX_MINOR
_CHUNK = 512
_GPC = _CHUNK // _IDX_MINOR
_G = _B_PER_W // _CHUNK
_T = _G // 2

_mesh = plsc.VectorSubcoreMesh(core_axis_name="c", subcore_axis_name="s")


@functools.partial(
    pl.kernel,
    mesh=_mesh,
    out_type=jax.ShapeDtypeStruct((_NW, _G, _CHUNK, _D), jnp.float32),
    scratch_types=[
        pltpu.VMEM((_IDX_ROWS_W, _IDX_MINOR), jnp.int32),
        pltpu.VMEM((_CHUNK, _D), jnp.float32),
        pltpu.VMEM((_CHUNK, _D), jnp.float32),
        pltpu.SemaphoreType.DMA,
        pltpu.SemaphoreType.DMA,
        pltpu.SemaphoreType.DMA,
        pltpu.SemaphoreType.DMA,
    ],
    compiler_params=pltpu.CompilerParams(use_tc_tiling_on_sc=False),
)
def _gather_kernel(table_hbm, idx_hbm, out_hbm, idx_v, rows0, rows1,
                   sem_g0, sem_g1, sem_s0, sem_s1):
    wid = lax.axis_index("s") * _NC + lax.axis_index("c")
    rows = (rows0, rows1)
    sem_g = (sem_g0, sem_g1)
    sem_s = (sem_s0, sem_s1)

    pltpu.sync_copy(idx_hbm.at[wid], idx_v)

    def fire_gathers(g, b):
        for j in range(_GPC):
            pltpu.async_copy(
                table_hbm.at[idx_v.at[g * _GPC + j]],
                rows[b].at[pl.ds(j * _IDX_MINOR, _IDX_MINOR)],
                sem_g[b],
            )

    def drain_gathers(b):
        pltpu.make_async_copy(
            table_hbm.at[pl.ds(0, _CHUNK)], rows[b], sem_g[b]
        ).wait()

    fire_gathers(0, 0)
    fire_gathers(1, 1)

    def body(t, carry):
        for b in range(2):
            g = 2 * t + b
            drain_gathers(b)
            store = pltpu.make_async_copy(rows[b], out_hbm.at[wid, g], sem_s[b])
            store.start()
            store.wait()

            @pl.when(t < _T - 1)
            def _():
                fire_gathers(g + 2, b)

        return carry

    lax.fori_loop(0, _T, body, 0)


def kernel(item_seq, ID_embeddings):
    idx = item_seq.astype(jnp.int32).reshape(_NW, _IDX_ROWS_W, _IDX_MINOR)
    out = _gather_kernel(ID_embeddings, idx)
    return out.reshape(_BATCH, _HIST, _D)
